# baseline (speedup 1.0000x reference)
import functools

import jax
import jax.numpy as jnp
from jax import lax
from jax.experimental import pallas as pl
from jax.experimental.pallas import tpu as pltpu
from jax.experimental.pallas import tpu_sc as plsc

NC, NS, L = 2, 16, 16
NW = NC * NS

_SC_PARAMS = pltpu.CompilerParams(use_tc_tiling_on_sc=False)


def _sc_mesh():
    return plsc.VectorSubcoreMesh(core_axis_name="c", subcore_axis_name="s")


def _make_deg_kernel(N, E, C):
    T = E // NW
    RS = N // NS

    @functools.partial(
        pl.kernel,
        out_type=jax.ShapeDtypeStruct((NC, N, L), jnp.float32),
        mesh=_sc_mesh(),
        scratch_types=[
            pltpu.VMEM((2, C), jnp.int32),
            pltpu.VMEM((C, L), jnp.float32),
            pltpu.VMEM_SHARED((N, L), jnp.float32),
            pltpu.SemaphoreType.DMA,
        ],
        compiler_params=_SC_PARAMS,
    )
    def deg_kernel(ei_hbm, zeros_hbm, ones_hbm, out_hbm, idx_v, ones_v, shared,
                   isem):
        ci = lax.axis_index("c")
        si = lax.axis_index("s")
        wid = si * NC + ci
        r0 = si * RS
        pltpu.sync_copy(zeros_hbm.at[pl.ds(r0, RS)], shared.at[pl.ds(r0, RS)])
        plsc.subcore_barrier()
        pltpu.sync_copy(ones_hbm, ones_v)
        ebase = E + wid * T
        K = T // C
        pltpu.async_copy(ei_hbm.at[pl.ds(ebase, C)], idx_v.at[0], isem)
        for k in range(K):
            b = k % 2
            pltpu.make_async_copy(ei_hbm.at[pl.ds(ebase, C)], idx_v.at[b], isem).wait()
            if k + 1 < K:
                pltpu.async_copy(
                    ei_hbm.at[pl.ds(ebase + (k + 1) * C, C)], idx_v.at[(k + 1) % 2], isem)
            pltpu.sync_copy(ones_v, shared.at[idx_v.at[b]], add=True)
        plsc.subcore_barrier()
        pltpu.sync_copy(shared.at[pl.ds(r0, RS)], out_hbm.at[ci, pl.ds(r0, RS)])

    return deg_kernel


def _make_agg_kernel(N, E, C):
    T = E // NW
    RS = N // NS

    @functools.partial(
        pl.kernel,
        out_type=jax.ShapeDtypeStruct((NC, N, L), jnp.float32),
        mesh=_sc_mesh(),
        scratch_types=[
            pltpu.VMEM((2, C), jnp.int32),
            pltpu.VMEM((2, C), jnp.int32),
            pltpu.VMEM((2, C, L), jnp.float32),
            pltpu.VMEM_SHARED((N, L), jnp.float32),
            pltpu.SemaphoreType.DMA,
        ],
        compiler_params=_SC_PARAMS,
    )
    def agg_kernel(ei_hbm, tab_hbm, zeros_hbm, out_hbm,
                   isrc_v, idst_v, rows_v, shared, gsem):
        ci = lax.axis_index("c")
        si = lax.axis_index("s")
        wid = si * NC + ci
        r0 = si * RS
        pltpu.sync_copy(zeros_hbm.at[pl.ds(r0, RS)], shared.at[pl.ds(r0, RS)])
        plsc.subcore_barrier()
        ebase = wid * T
        K = T // C
        pltpu.sync_copy(ei_hbm.at[pl.ds(ebase, C)], isrc_v.at[0])
        pltpu.sync_copy(ei_hbm.at[pl.ds(E + ebase, C)], idst_v.at[0])
        pltpu.async_copy(tab_hbm.at[isrc_v.at[0]], rows_v.at[0], gsem)
        for k in range(K):
            b = k % 2
            nb = (k + 1) % 2
            if k + 1 < K:
                e1 = ebase + (k + 1) * C
                pltpu.sync_copy(ei_hbm.at[pl.ds(e1, C)], isrc_v.at[nb])
                pltpu.sync_copy(ei_hbm.at[pl.ds(E + e1, C)], idst_v.at[nb])
            pltpu.make_async_copy(tab_hbm.at[isrc_v.at[b]], rows_v.at[b], gsem).wait()
            if k + 1 < K:
                pltpu.async_copy(tab_hbm.at[isrc_v.at[nb]], rows_v.at[nb], gsem)
            pltpu.sync_copy(rows_v.at[b], shared.at[idst_v.at[b]], add=True)
        plsc.subcore_barrier()
        pltpu.sync_copy(shared.at[pl.ds(r0, RS)], out_hbm.at[ci, pl.ds(r0, RS)])

    return agg_kernel


def _make_agg_gather_kernel(N, E, C):
    T = E // NW
    RS = N // NS

    @functools.partial(
        pl.kernel,
        out_type=(
            jax.ShapeDtypeStruct((NC, N, L), jnp.float32),
            jax.ShapeDtypeStruct((E, L), jnp.bfloat16),
            jax.ShapeDtypeStruct((E, L), jnp.bfloat16),
        ),
        mesh=_sc_mesh(),
        scratch_types=[
            pltpu.VMEM((2, C), jnp.int32),
            pltpu.VMEM((2, C), jnp.int32),
            pltpu.VMEM((2, C, L), jnp.float32),
            pltpu.VMEM((2, C, L), jnp.bfloat16),
            pltpu.VMEM((2, C, L), jnp.bfloat16),
            pltpu.VMEM_SHARED((N, L), jnp.float32),
            pltpu.SemaphoreType.DMA,
            pltpu.SemaphoreType.DMA,
            pltpu.SemaphoreType.DMA,
            pltpu.SemaphoreType.DMA,
            pltpu.SemaphoreType.DMA,
            pltpu.SemaphoreType.DMA,
        ],
        compiler_params=_SC_PARAMS,
    )
    def aggg_kernel(ei_hbm, g_hbm, h_hbm, zeros_hbm,
                    acc_hbm, hs_hbm, hd_hbm,
                    isrc_v, idst_v, grows_v, hsrows_v, hdrows_v, shared,
                    ga0, ga1, gb0, gb1, gc0, gc1):
        ci = lax.axis_index("c")
        si = lax.axis_index("s")
        wid = si * NC + ci
        r0 = si * RS
        gsa = (ga0, ga1)
        gsb = (gb0, gb1)
        gsc = (gc0, gc1)
        pltpu.sync_copy(zeros_hbm.at[pl.ds(r0, RS)], shared.at[pl.ds(r0, RS)])
        plsc.subcore_barrier()
        ebase = wid * T
        K = T // C

        def issue(k):
            b = k % 2
            e0 = ebase + k * C
            pltpu.sync_copy(ei_hbm.at[pl.ds(e0, C)], isrc_v.at[b])
            pltpu.sync_copy(ei_hbm.at[pl.ds(E + e0, C)], idst_v.at[b])
            pltpu.async_copy(g_hbm.at[isrc_v.at[b]], grows_v.at[b], gsa[b])
            pltpu.async_copy(h_hbm.at[isrc_v.at[b]], hsrows_v.at[b], gsb[b])
            pltpu.async_copy(h_hbm.at[idst_v.at[b]], hdrows_v.at[b], gsc[b])

        issue(0)
        for k in range(K):
            b = k % 2
            if k + 1 < K:
                issue(k + 1)
            e0 = ebase + k * C
            pltpu.make_async_copy(g_hbm.at[isrc_v.at[b]], grows_v.at[b], gsa[b]).wait()
            pltpu.sync_copy(grows_v.at[b], shared.at[idst_v.at[b]], add=True)
            pltpu.make_async_copy(h_hbm.at[isrc_v.at[b]], hsrows_v.at[b], gsb[b]).wait()
            pltpu.sync_copy(hsrows_v.at[b], hs_hbm.at[pl.ds(e0, C)])
            pltpu.make_async_copy(h_hbm.at[idst_v.at[b]], hdrows_v.at[b], gsc[b]).wait()
            pltpu.sync_copy(hdrows_v.at[b], hd_hbm.at[pl.ds(e0, C)])
        plsc.subcore_barrier()
        pltpu.sync_copy(shared.at[pl.ds(r0, RS)], acc_hbm.at[ci, pl.ds(r0, RS)])

    return aggg_kernel


def _k1a_body(x_ref, w1_ref, h0_ref):
    h0 = jnp.dot(x_ref[...], w1_ref[...], preferred_element_type=jnp.float32)
    h0_ref[...] = h0


def _make_k1b_body(NL):
    def _k1b_body(h0_ref, degp_ref, g0_ref, dinv_ref):
        deg = degp_ref[pl.ds(0, NL)] + degp_ref[pl.ds(NL, NL)] + 1.0
        dinv = lax.rsqrt(deg)
        dinv_ref[...] = dinv
        g0_ref[...] = h0_ref[...] * dinv
    return _k1b_body


def _make_k2_body(NL):
    def _k2_body(accp_ref, g0_ref, dinv_ref, b1t_ref, h_ref, g_ref):
        dinv = dinv_ref[...]
        acc = accp_ref[pl.ds(0, NL)] + accp_ref[pl.ds(NL, NL)] + g0_ref[...]
        h = jnp.maximum(dinv * acc + b1t_ref[...], 0.0)
        h_ref[...] = h.astype(jnp.bfloat16)
        g_ref[...] = h * dinv
    return _k2_body


def _make_k3_body(RB, NL, G):
    NR = NL // 128

    def _k3_body(hs_ref, hd_ref, wa_ref, wb_ref, bl1_ref, wg_ref, bl2_ref,
                 accp_ref, g_ref, dinv_ref, w2k_ref, b2t_ref,
                 edge_ref, node_ref):
        i = pl.program_id(0)

        @pl.when(i == 0)
        def _():
            acc = accp_ref[pl.ds(0, NL)] + accp_ref[pl.ds(NL, NL)] + g_ref[...]
            aggh = dinv_ref[...] * acc
            m = aggh.reshape(NR, 128)
            node_ref[...] = (
                jnp.dot(m, w2k_ref[...], preferred_element_type=jnp.float32)
                + b2t_ref[...][None, :]
            )
        e = (
            jnp.dot(hs_ref[...], wa_ref[...], preferred_element_type=jnp.float32)
            + jnp.dot(hd_ref[...], wb_ref[...], preferred_element_type=jnp.float32)
            + bl1_ref[...]
        )
        e = jnp.maximum(e, 0.0)
        z = jnp.dot(e, wg_ref[...], preferred_element_type=jnp.float32) + bl2_ref[0]
        edge_ref[...] = jax.nn.sigmoid(z)
    return _k3_body


def kernel(x, edge_index, W1, b1, W2, b2, Wl1, bl1, Wl2, bl2):
    N, D = x.shape
    E = edge_index.shape[1]
    H1 = W1.shape[1]
    assert H1 == L and N % NS == 0 and E % NW == 0
    C = 2000
    NL = N * L
    ei_flat = edge_index.reshape(2 * E)
    zeros = jnp.zeros((N, L), jnp.float32)
    ones = jnp.ones((C, L), jnp.float32)

    h0 = pl.pallas_call(
        _k1a_body,
        out_shape=jax.ShapeDtypeStruct((N, L), jnp.float32),
    )(x, W1)

    degp = _make_deg_kernel(N, E, C)(ei_flat, zeros, ones)

    g0_1d, dinv_1d = pl.pallas_call(
        _make_k1b_body(NL),
        out_shape=(
            jax.ShapeDtypeStruct((NL,), jnp.float32),
            jax.ShapeDtypeStruct((NL,), jnp.float32),
        ),
    )(h0.reshape(NL), degp.reshape(2 * NL))

    accp1 = _make_agg_kernel(N, E, C)(ei_flat, g0_1d.reshape(N, L), zeros)

    b1t = jnp.tile(b1, N)
    h_1d, g_1d = pl.pallas_call(
        _make_k2_body(NL),
        out_shape=(
            jax.ShapeDtypeStruct((NL,), jnp.bfloat16),
            jax.ShapeDtypeStruct((NL,), jnp.float32),
        ),
    )(accp1.reshape(2 * NL), g0_1d, dinv_1d, b1t)

    accp2, hs, hd = _make_agg_gather_kernel(N, E, 1000)(
        ei_flat, g_1d.reshape(N, L), h_1d.reshape(N, L), zeros)

    G = 8
    R = E // G
    STEPS = 10
    RB = R // STEPS
    NB = N // STEPS
    NBL = NB * L
    hs8 = hs.reshape(R, G * L)
    hd8 = hd.reshape(R, G * L)
    eye8 = jnp.eye(G, dtype=jnp.float32)
    wa = jnp.kron(eye8, Wl1[:L]).astype(jnp.bfloat16)
    wb = jnp.kron(eye8, Wl1[L:]).astype(jnp.bfloat16)
    bl1_t = jnp.tile(bl1, G)[None, :]
    wg = jnp.kron(eye8, Wl2)
    H2 = W2.shape[1]
    w2k = jnp.kron(eye8, W2)
    b2t = jnp.tile(b2, G)
    accp2_1d = accp2.reshape(2 * NL)
    edge_out, node_pk = pl.pallas_call(
        _make_k3_body(RB, NL, G),
        grid=(STEPS,),
        in_specs=[
            pl.BlockSpec((RB, G * L), lambda i: (i, 0)),
            pl.BlockSpec((RB, G * L), lambda i: (i, 0)),
            pl.BlockSpec(wa.shape, lambda i: (0, 0)),
            pl.BlockSpec(wb.shape, lambda i: (0, 0)),
            pl.BlockSpec(bl1_t.shape, lambda i: (0, 0)),
            pl.BlockSpec(wg.shape, lambda i: (0, 0)),
            pl.BlockSpec(bl2.shape, lambda i: (0,)),
            pl.BlockSpec((2 * NL,), lambda i: (0,)),
            pl.BlockSpec((NL,), lambda i: (0,)),
            pl.BlockSpec((NL,), lambda i: (0,)),
            pl.BlockSpec(w2k.shape, lambda i: (0, 0)),
            pl.BlockSpec(b2t.shape, lambda i: (0,)),
        ],
        out_specs=(
            pl.BlockSpec((RB, G), lambda i: (i, 0)),
            pl.BlockSpec((NL // 128, G * H2), lambda i: (0, 0)),
        ),
        out_shape=(
            jax.ShapeDtypeStruct((R, G), jnp.float32),
            jax.ShapeDtypeStruct((NL // 128, G * H2), jnp.float32),
        ),
    )(hs8, hd8, wa, wb, bl1_t, wg, bl2,
      accp2_1d, g_1d, dinv_1d, w2k, b2t)

    return node_pk.reshape(N, H2), edge_out.reshape(E)


# speedup vs baseline: 1.5156x; 1.5156x over previous
import functools

import jax
import jax.numpy as jnp
from jax import lax
from jax.experimental import pallas as pl
from jax.experimental.pallas import tpu as pltpu
from jax.experimental.pallas import tpu_sc as plsc

NC, NS, L = 2, 16, 16
NW = NC * NS

_SC_PARAMS = pltpu.CompilerParams(use_tc_tiling_on_sc=False)


def _sc_mesh():
    return plsc.VectorSubcoreMesh(core_axis_name="c", subcore_axis_name="s")


def _make_deg_kernel(N, E, C):
    T = E // NW
    RS = N // NS

    @functools.partial(
        pl.kernel,
        out_type=jax.ShapeDtypeStruct((NC, N, L), jnp.float32),
        mesh=_sc_mesh(),
        scratch_types=[
            pltpu.VMEM((2, C), jnp.int32),
            pltpu.VMEM((C, L), jnp.float32),
            pltpu.VMEM_SHARED((N, L), jnp.float32),
            pltpu.SemaphoreType.DMA,
        ],
        compiler_params=_SC_PARAMS,
    )
    def deg_kernel(ei_hbm, zeros_hbm, ones_hbm, out_hbm, idx_v, ones_v, shared,
                   isem):
        ci = lax.axis_index("c")
        si = lax.axis_index("s")
        wid = si * NC + ci
        r0 = si * RS
        pltpu.sync_copy(zeros_hbm.at[pl.ds(r0, RS)], shared.at[pl.ds(r0, RS)])
        plsc.subcore_barrier()
        pltpu.sync_copy(ones_hbm, ones_v)
        ebase = E + wid * T
        K = T // C
        pltpu.async_copy(ei_hbm.at[pl.ds(ebase, C)], idx_v.at[0], isem)
        for k in range(K):
            b = k % 2
            pltpu.make_async_copy(ei_hbm.at[pl.ds(ebase, C)], idx_v.at[b], isem).wait()
            if k + 1 < K:
                pltpu.async_copy(
                    ei_hbm.at[pl.ds(ebase + (k + 1) * C, C)], idx_v.at[(k + 1) % 2], isem)
            pltpu.sync_copy(ones_v, shared.at[idx_v.at[b]], add=True)
        plsc.subcore_barrier()
        pltpu.sync_copy(shared.at[pl.ds(r0, RS)], out_hbm.at[ci, pl.ds(r0, RS)])

    return deg_kernel


def _make_agg_kernel(N, E, C):
    T = E // NW
    RS = N // NS

    @functools.partial(
        pl.kernel,
        out_type=jax.ShapeDtypeStruct((NC, N, L), jnp.float32),
        mesh=_sc_mesh(),
        scratch_types=[
            pltpu.VMEM((2, C), jnp.int32),
            pltpu.VMEM((2, C), jnp.int32),
            pltpu.VMEM((2, C, L), jnp.float32),
            pltpu.VMEM_SHARED((N, L), jnp.float32),
            pltpu.SemaphoreType.DMA,
        ],
        compiler_params=_SC_PARAMS,
    )
    def agg_kernel(ei_hbm, tab_hbm, zeros_hbm, out_hbm,
                   isrc_v, idst_v, rows_v, shared, gsem):
        ci = lax.axis_index("c")
        si = lax.axis_index("s")
        wid = si * NC + ci
        r0 = si * RS
        pltpu.sync_copy(zeros_hbm.at[pl.ds(r0, RS)], shared.at[pl.ds(r0, RS)])
        plsc.subcore_barrier()
        ebase = wid * T
        K = T // C
        pltpu.sync_copy(ei_hbm.at[pl.ds(ebase, C)], isrc_v.at[0])
        pltpu.sync_copy(ei_hbm.at[pl.ds(E + ebase, C)], idst_v.at[0])
        pltpu.async_copy(tab_hbm.at[isrc_v.at[0]], rows_v.at[0], gsem)
        for k in range(K):
            b = k % 2
            nb = (k + 1) % 2
            if k + 1 < K:
                e1 = ebase + (k + 1) * C
                pltpu.sync_copy(ei_hbm.at[pl.ds(e1, C)], isrc_v.at[nb])
                pltpu.sync_copy(ei_hbm.at[pl.ds(E + e1, C)], idst_v.at[nb])
            pltpu.make_async_copy(tab_hbm.at[isrc_v.at[b]], rows_v.at[b], gsem).wait()
            if k + 1 < K:
                pltpu.async_copy(tab_hbm.at[isrc_v.at[nb]], rows_v.at[nb], gsem)
            pltpu.sync_copy(rows_v.at[b], shared.at[idst_v.at[b]], add=True)
        plsc.subcore_barrier()
        pltpu.sync_copy(shared.at[pl.ds(r0, RS)], out_hbm.at[ci, pl.ds(r0, RS)])

    return agg_kernel


def _make_agg_gather_kernel(N, E, C):
    T = E // NW
    RS = N // NS

    @functools.partial(
        pl.kernel,
        out_type=(
            jax.ShapeDtypeStruct((NC, N, L), jnp.float32),
            jax.ShapeDtypeStruct((E, L), jnp.float32),
            jax.ShapeDtypeStruct((E, L), jnp.float32),
        ),
        mesh=_sc_mesh(),
        scratch_types=[
            pltpu.VMEM((2, C), jnp.int32),
            pltpu.VMEM((2, C), jnp.int32),
            pltpu.VMEM((2, C, L), jnp.float32),
            pltpu.VMEM((2, C, L), jnp.float32),
            pltpu.VMEM((2, C, L), jnp.float32),
            pltpu.VMEM_SHARED((N, L), jnp.float32),
            pltpu.SemaphoreType.DMA,
            pltpu.SemaphoreType.DMA,
            pltpu.SemaphoreType.DMA,
            pltpu.SemaphoreType.DMA,
            pltpu.SemaphoreType.DMA,
            pltpu.SemaphoreType.DMA,
        ],
        compiler_params=_SC_PARAMS,
    )
    def aggg_kernel(ei_hbm, g_hbm, h_hbm, zeros_hbm,
                    acc_hbm, hs_hbm, hd_hbm,
                    isrc_v, idst_v, grows_v, hsrows_v, hdrows_v, shared,
                    ga0, ga1, gb0, gb1, gc0, gc1):
        ci = lax.axis_index("c")
        si = lax.axis_index("s")
        wid = si * NC + ci
        r0 = si * RS
        gsa = (ga0, ga1)
        gsb = (gb0, gb1)
        gsc = (gc0, gc1)
        pltpu.sync_copy(zeros_hbm.at[pl.ds(r0, RS)], shared.at[pl.ds(r0, RS)])
        plsc.subcore_barrier()
        ebase = wid * T
        K = T // C

        def issue(k):
            b = k % 2
            e0 = ebase + k * C
            pltpu.sync_copy(ei_hbm.at[pl.ds(e0, C)], isrc_v.at[b])
            pltpu.sync_copy(ei_hbm.at[pl.ds(E + e0, C)], idst_v.at[b])
            pltpu.async_copy(g_hbm.at[isrc_v.at[b]], grows_v.at[b], gsa[b])
            pltpu.async_copy(h_hbm.at[isrc_v.at[b]], hsrows_v.at[b], gsb[b])
            pltpu.async_copy(h_hbm.at[idst_v.at[b]], hdrows_v.at[b], gsc[b])

        issue(0)
        for k in range(K):
            b = k % 2
            if k + 1 < K:
                issue(k + 1)
            e0 = ebase + k * C
            pltpu.make_async_copy(g_hbm.at[isrc_v.at[b]], grows_v.at[b], gsa[b]).wait()
            pltpu.sync_copy(grows_v.at[b], shared.at[idst_v.at[b]], add=True)
            pltpu.make_async_copy(h_hbm.at[isrc_v.at[b]], hsrows_v.at[b], gsb[b]).wait()
            pltpu.sync_copy(hsrows_v.at[b], hs_hbm.at[pl.ds(e0, C)])
            pltpu.make_async_copy(h_hbm.at[idst_v.at[b]], hdrows_v.at[b], gsc[b]).wait()
            pltpu.sync_copy(hdrows_v.at[b], hd_hbm.at[pl.ds(e0, C)])
        plsc.subcore_barrier()
        pltpu.sync_copy(shared.at[pl.ds(r0, RS)], acc_hbm.at[ci, pl.ds(r0, RS)])

    return aggg_kernel


def _k1a_body(x_ref, w1_ref, h0_ref):
    h0 = jnp.dot(x_ref[...], w1_ref[...], preferred_element_type=jnp.float32)
    h0_ref[...] = h0


def _make_k1b_body(NL):
    def _k1b_body(h0_ref, degp_ref, g0_ref, dinv_ref):
        deg = degp_ref[pl.ds(0, NL)] + degp_ref[pl.ds(NL, NL)] + 1.0
        dinv = lax.rsqrt(deg)
        dinv_ref[...] = dinv
        g0_ref[...] = h0_ref[...] * dinv
    return _k1b_body


def _make_k2_body(NL):
    def _k2_body(accp_ref, g0_ref, dinv_ref, b1t_ref, h_ref, g_ref):
        dinv = dinv_ref[...]
        acc = accp_ref[pl.ds(0, NL)] + accp_ref[pl.ds(NL, NL)] + g0_ref[...]
        h = jnp.maximum(dinv * acc + b1t_ref[...], 0.0)
        h_ref[...] = h
        g_ref[...] = h * dinv
    return _k2_body


def _make_k3_body(RB, NL, G):
    NR = NL // 128

    def _k3_body(hs_ref, hd_ref, wa_ref, wb_ref, bl1_ref, wg_ref, bl2_ref,
                 accp_ref, g_ref, dinv_ref, w2k_ref, b2t_ref,
                 edge_ref, node_ref):
        i = pl.program_id(0)

        @pl.when(i == 0)
        def _():
            acc = accp_ref[pl.ds(0, NL)] + accp_ref[pl.ds(NL, NL)] + g_ref[...]
            aggh = dinv_ref[...] * acc
            m = aggh.reshape(NR, 128)
            node_ref[...] = (
                jnp.dot(m, w2k_ref[...], preferred_element_type=jnp.float32)
                + b2t_ref[...][None, :]
            )
        e = (
            jnp.dot(hs_ref[...], wa_ref[...], preferred_element_type=jnp.float32)
            + jnp.dot(hd_ref[...], wb_ref[...], preferred_element_type=jnp.float32)
            + bl1_ref[...]
        )
        e = jnp.maximum(e, 0.0)
        z = jnp.dot(e, wg_ref[...], preferred_element_type=jnp.float32) + bl2_ref[0]
        edge_ref[...] = jax.nn.sigmoid(z)
    return _k3_body


def kernel(x, edge_index, W1, b1, W2, b2, Wl1, bl1, Wl2, bl2):
    N, D = x.shape
    E = edge_index.shape[1]
    H1 = W1.shape[1]
    assert H1 == L and N % NS == 0 and E % NW == 0
    C = 2000
    NL = N * L
    ei_flat = edge_index.reshape(2 * E)
    zeros = jnp.zeros((N, L), jnp.float32)
    ones = jnp.ones((C, L), jnp.float32)

    h0 = pl.pallas_call(
        _k1a_body,
        out_shape=jax.ShapeDtypeStruct((N, L), jnp.float32),
    )(x, W1)

    degp = _make_deg_kernel(N, E, C)(ei_flat, zeros, ones)

    g0_1d, dinv_1d = pl.pallas_call(
        _make_k1b_body(NL),
        out_shape=(
            jax.ShapeDtypeStruct((NL,), jnp.float32),
            jax.ShapeDtypeStruct((NL,), jnp.float32),
        ),
    )(h0.reshape(NL), degp.reshape(2 * NL))

    accp1 = _make_agg_kernel(N, E, C)(ei_flat, g0_1d.reshape(N, L), zeros)

    b1t = jnp.tile(b1, N)
    h_1d, g_1d = pl.pallas_call(
        _make_k2_body(NL),
        out_shape=(
            jax.ShapeDtypeStruct((NL,), jnp.float32),
            jax.ShapeDtypeStruct((NL,), jnp.float32),
        ),
    )(accp1.reshape(2 * NL), g0_1d, dinv_1d, b1t)

    accp2, hs, hd = _make_agg_gather_kernel(N, E, 1000)(
        ei_flat, g_1d.reshape(N, L), h_1d.reshape(N, L), zeros)

    G = 8
    R = E // G
    STEPS = 10
    RB = R // STEPS
    NB = N // STEPS
    NBL = NB * L
    hs8 = hs.reshape(R, G * L)
    hd8 = hd.reshape(R, G * L)
    eye8 = jnp.eye(G, dtype=jnp.float32)
    wa = jnp.kron(eye8, Wl1[:L])
    wb = jnp.kron(eye8, Wl1[L:])
    bl1_t = jnp.tile(bl1, G)[None, :]
    wg = jnp.kron(eye8, Wl2)
    H2 = W2.shape[1]
    w2k = jnp.kron(eye8, W2)
    b2t = jnp.tile(b2, G)
    accp2_1d = accp2.reshape(2 * NL)
    edge_out, node_pk = pl.pallas_call(
        _make_k3_body(RB, NL, G),
        grid=(STEPS,),
        in_specs=[
            pl.BlockSpec((RB, G * L), lambda i: (i, 0)),
            pl.BlockSpec((RB, G * L), lambda i: (i, 0)),
            pl.BlockSpec(wa.shape, lambda i: (0, 0)),
            pl.BlockSpec(wb.shape, lambda i: (0, 0)),
            pl.BlockSpec(bl1_t.shape, lambda i: (0, 0)),
            pl.BlockSpec(wg.shape, lambda i: (0, 0)),
            pl.BlockSpec(bl2.shape, lambda i: (0,)),
            pl.BlockSpec((2 * NL,), lambda i: (0,)),
            pl.BlockSpec((NL,), lambda i: (0,)),
            pl.BlockSpec((NL,), lambda i: (0,)),
            pl.BlockSpec(w2k.shape, lambda i: (0, 0)),
            pl.BlockSpec(b2t.shape, lambda i: (0,)),
        ],
        out_specs=(
            pl.BlockSpec((RB, G), lambda i: (i, 0)),
            pl.BlockSpec((NL // 128, G * H2), lambda i: (0, 0)),
        ),
        out_shape=(
            jax.ShapeDtypeStruct((R, G), jnp.float32),
            jax.ShapeDtypeStruct((NL // 128, G * H2), jnp.float32),
        ),
    )(hs8, hd8, wa, wb, bl1_t, wg, bl2,
      accp2_1d, g_1d, dinv_1d, w2k, b2t)

    return node_pk.reshape(N, H2), edge_out.reshape(E)
